# baseline (speedup 1.0000x reference)
import functools

import jax
import jax.numpy as jnp
from jax import lax
from jax.experimental import pallas as pl
from jax.experimental.pallas import tpu as pltpu
from jax.experimental.pallas import tpu_sc as plsc

_VOCAB = 1000000
_D = 64
_B = 1024
_S = 200
_F = 4

_NC = 2
_NS = 16
_NW = _NC * _NS
_TOTAL = _B * _S * _F
_PER_W = _TOTAL // _NW
_CHUNK = 128
_NBUF = 5
_STEPS = _PER_W // _CHUNK
_G = _STEPS // _NBUF

_BW = _B // _NW
_SR = _S // 8


_TBLK = 512
_KPAIR = 500224
_NTB = _KPAIR // _TBLK


def _tp_body(ta_ref, tb_ref, out_ref):
    out_ref[:, 0:_D] = ta_ref[...].T
    out_ref[:, _D:128] = tb_ref[...].T


_transpose_table = pl.pallas_call(
    _tp_body,
    grid=(_NTB,),
    in_specs=[
        pl.BlockSpec((_D, _TBLK), lambda i: (0, i)),
        pl.BlockSpec((_D, _TBLK), lambda i: (0, i + _NTB)),
    ],
    out_specs=pl.BlockSpec((_TBLK, 128), lambda i: (i, 0)),
    out_shape=jax.ShapeDtypeStruct((_KPAIR, 128), jnp.float32),
)


_mesh = plsc.VectorSubcoreMesh(core_axis_name="c", subcore_axis_name="s")


@functools.partial(
    pl.kernel,
    mesh=_mesh,
    out_type=jax.ShapeDtypeStruct((_TOTAL, _D), jnp.float32),
    scratch_types=[
        pltpu.VMEM((_SR, 8, _BW), jnp.int32),
        pltpu.VMEM((_SR, 8, _BW), jnp.int32),
        pltpu.VMEM((_SR, 8, _BW), jnp.int32),
        pltpu.VMEM((_SR, 8, _BW), jnp.int32),
        pltpu.VMEM((_PER_W,), jnp.int32),
        pltpu.VMEM((_NBUF, _CHUNK, _D), jnp.float32),
        pltpu.SemaphoreType.DMA((_NBUF,)),
        pltpu.SemaphoreType.DMA((_NBUF,)),
    ],
    compiler_params=pltpu.CompilerParams(use_tc_tiling_on_sc=False),
)
def _lookup(table_hbm, l0, l1, l2, l3, out_hbm,
            f0v, f1v, f2v, f3v, glist, rows_v, gsem, osem):
    wid = lax.axis_index("s") * _NC + lax.axis_index("c")
    base = wid * _PER_W
    cblk = wid // 4
    coff = (wid % 4) * _BW

    for l_hbm, f_v in ((l0, f0v), (l1, f1v), (l2, f2v), (l3, f3v)):
        pltpu.sync_copy(l_hbm.at[:, cblk, :, pl.ds(coff, _BW)], f_v)

    iot = lax.iota(jnp.int32, 16)
    lane_e = [iot == (2 * u) for u in range(8)]
    lane_o = [iot == (2 * u + 1) for u in range(8)]

    def rb_loop(t, carry):
        r = lax.shift_right_logical(t, 1)
        bh16 = (t & 1) * 16
        for cp in range(2):
            fe, fo = (f0v, f1v) if cp == 0 else (f2v, f3v)
            ve = [fe[r, u, pl.ds(bh16, 16)] for u in range(8)]
            vo = [fo[r, u, pl.ds(bh16, 16)] for u in range(8)]
            for bl in range(16):
                cidx = jnp.full((16,), bl, jnp.int32)
                acc = jnp.zeros((16,), jnp.int32)
                for u in range(8):
                    acc = jnp.where(lane_e[u], ve[u][cidx], acc)
                    acc = jnp.where(lane_o[u], vo[u][cidx], acc)
                g = acc + acc - jnp.where(acc >= _KPAIR, 2 * _KPAIR - 1, 0)
                off = ((bh16 + bl) * _SR + r) * 32 + cp * 16
                glist[pl.ds(off, 16)] = g
        return carry

    lax.fori_loop(0, 2 * _SR, rb_loop, 0)

    def fire_gather(j, slot):
        pltpu.async_copy(
            table_hbm.at[glist.at[pl.ds(j * _CHUNK, _CHUNK)]],
            rows_v.at[slot],
            gsem.at[slot],
        )

    def wait_gather(slot):
        pltpu.make_async_copy(
            table_hbm.at[glist.at[pl.ds(0, _CHUNK)]],
            rows_v.at[slot],
            gsem.at[slot],
        ).wait()

    def fire_store(j, slot):
        pltpu.async_copy(
            rows_v.at[slot],
            out_hbm.at[pl.ds(base + j * _CHUNK, _CHUNK)],
            osem.at[slot],
        )

    def wait_store(slot):
        pltpu.make_async_copy(
            rows_v.at[slot],
            out_hbm.at[pl.ds(base, _CHUNK)],
            osem.at[slot],
        ).wait()

    for b in range(_NBUF - 1):
        fire_gather(b, b)

    def run_step(j, b, first, fire):
        wait_gather(b)
        fire_store(j, b)
        if not first:
            wait_store((b - 1) % _NBUF)
        if fire:
            fire_gather(j + _NBUF - 1, (b - 1) % _NBUF)

    for b in range(_NBUF):
        run_step(b, b, first=(b == 0), fire=True)

    def outer(g, carry):
        for b in range(_NBUF):
            run_step(g * _NBUF + b, b, first=False, fire=True)
        return carry

    lax.fori_loop(1, _G - 1, outer, 0)

    for b in range(_NBUF):
        j = (_G - 1) * _NBUF + b
        run_step(j, b, first=False, fire=(j + _NBUF - 1 <= _STEPS - 1))
    wait_store(_NBUF - 1)


def kernel(table, idx_f0, idx_f1, idx_f2, idx_f3, seq_lens):
    del seq_lens
    tt = table.T
    tlin = _transpose_table(tt, tt).reshape(2 * _KPAIR, _D)
    ls = [
        f.T.reshape(_SR, 8, _B // 128, 128).transpose(0, 2, 1, 3)
        for f in (idx_f0, idx_f1, idx_f2, idx_f3)
    ]
    rows = _lookup(tlin, *ls)
    return (
        rows.reshape(_B, _SR, _F // 2, 8, 2 * _D)
        .transpose(0, 1, 3, 2, 4)
        .reshape(_B, _S, _F * _D)
    )


# speedup vs baseline: 1.7345x; 1.7345x over previous
import functools

import jax
import jax.numpy as jnp
from jax import lax
from jax.experimental import pallas as pl
from jax.experimental.pallas import tpu as pltpu
from jax.experimental.pallas import tpu_sc as plsc

_VOCAB = 1000000
_D = 64
_B = 1024
_S = 200
_F = 4

_NC = 2
_NS = 16
_NW = _NC * _NS
_TOTAL = _B * _S * _F
_PER_W = _TOTAL // _NW
_CHUNK = 128
_NBUF = 5
_STEPS = _PER_W // _CHUNK
_G = _STEPS // _NBUF

_BW = _B // _NW
_SR = _S // 8


_TBLK = 2048
_KPAIR = 501760
_NTB = _KPAIR // _TBLK


def _tp_body(ta_ref, tb_ref, out_ref):
    out_ref[:, 0:_D] = ta_ref[...].T
    out_ref[:, _D:128] = tb_ref[...].T


_transpose_table = pl.pallas_call(
    _tp_body,
    grid=(_NTB,),
    in_specs=[
        pl.BlockSpec((_D, _TBLK), lambda i: (0, i)),
        pl.BlockSpec(
            (_D, _TBLK),
            lambda i: (0, jnp.minimum(i + _NTB, (_VOCAB - 1) // _TBLK)),
        ),
    ],
    out_specs=pl.BlockSpec((_TBLK, 128), lambda i: (i, 0)),
    out_shape=jax.ShapeDtypeStruct((_KPAIR, 128), jnp.float32),
)


_mesh = plsc.VectorSubcoreMesh(core_axis_name="c", subcore_axis_name="s")


@functools.partial(
    pl.kernel,
    mesh=_mesh,
    out_type=jax.ShapeDtypeStruct((_TOTAL, _D), jnp.float32),
    scratch_types=[
        pltpu.VMEM((_SR, 8, _BW), jnp.int32),
        pltpu.VMEM((_SR, 8, _BW), jnp.int32),
        pltpu.VMEM((_SR, 8, _BW), jnp.int32),
        pltpu.VMEM((_SR, 8, _BW), jnp.int32),
        pltpu.VMEM((_PER_W,), jnp.int32),
        pltpu.VMEM((_NBUF, _CHUNK, _D), jnp.float32),
        pltpu.SemaphoreType.DMA((_NBUF,)),
        pltpu.SemaphoreType.DMA((_NBUF,)),
    ],
    compiler_params=pltpu.CompilerParams(use_tc_tiling_on_sc=False),
)
def _lookup(table_hbm, l0, l1, l2, l3, out_hbm,
            f0v, f1v, f2v, f3v, glist, rows_v, gsem, osem):
    wid = lax.axis_index("s") * _NC + lax.axis_index("c")
    base = wid * _PER_W
    cblk = wid // 4
    coff = (wid % 4) * _BW

    for l_hbm, f_v in ((l0, f0v), (l1, f1v), (l2, f2v), (l3, f3v)):
        pltpu.sync_copy(l_hbm.at[:, cblk, :, pl.ds(coff, _BW)], f_v)

    iot = lax.iota(jnp.int32, 16)
    lane_e = [iot == (2 * u) for u in range(8)]
    lane_o = [iot == (2 * u + 1) for u in range(8)]

    def rb_loop(t, carry):
        r = lax.shift_right_logical(t, 1)
        bh16 = (t & 1) * 16
        for cp in range(2):
            fe, fo = (f0v, f1v) if cp == 0 else (f2v, f3v)
            ve = [fe[r, u, pl.ds(bh16, 16)] for u in range(8)]
            vo = [fo[r, u, pl.ds(bh16, 16)] for u in range(8)]
            for bl in range(16):
                cidx = jnp.full((16,), bl, jnp.int32)
                acc = jnp.zeros((16,), jnp.int32)
                for u in range(8):
                    acc = jnp.where(lane_e[u], ve[u][cidx], acc)
                    acc = jnp.where(lane_o[u], vo[u][cidx], acc)
                g = acc + acc - jnp.where(acc >= _KPAIR, 2 * _KPAIR - 1, 0)
                off = ((bh16 + bl) * _SR + r) * 32 + cp * 16
                glist[pl.ds(off, 16)] = g
        return carry

    lax.fori_loop(0, 2 * _SR, rb_loop, 0)

    def fire_gather(j, slot):
        pltpu.async_copy(
            table_hbm.at[glist.at[pl.ds(j * _CHUNK, _CHUNK)]],
            rows_v.at[slot],
            gsem.at[slot],
        )

    def wait_gather(slot):
        pltpu.make_async_copy(
            table_hbm.at[glist.at[pl.ds(0, _CHUNK)]],
            rows_v.at[slot],
            gsem.at[slot],
        ).wait()

    def fire_store(j, slot):
        pltpu.async_copy(
            rows_v.at[slot],
            out_hbm.at[pl.ds(base + j * _CHUNK, _CHUNK)],
            osem.at[slot],
        )

    def wait_store(slot):
        pltpu.make_async_copy(
            rows_v.at[slot],
            out_hbm.at[pl.ds(base, _CHUNK)],
            osem.at[slot],
        ).wait()

    for b in range(_NBUF - 1):
        fire_gather(b, b)

    def run_step(j, b, first, fire):
        wait_gather(b)
        fire_store(j, b)
        if not first:
            wait_store((b - 1) % _NBUF)
        if fire:
            fire_gather(j + _NBUF - 1, (b - 1) % _NBUF)

    for b in range(_NBUF):
        run_step(b, b, first=(b == 0), fire=True)

    def outer(g, carry):
        for b in range(_NBUF):
            run_step(g * _NBUF + b, b, first=False, fire=True)
        return carry

    lax.fori_loop(1, _G - 1, outer, 0)

    for b in range(_NBUF):
        j = (_G - 1) * _NBUF + b
        run_step(j, b, first=False, fire=(j + _NBUF - 1 <= _STEPS - 1))
    wait_store(_NBUF - 1)


def kernel(table, idx_f0, idx_f1, idx_f2, idx_f3, seq_lens):
    del seq_lens
    tt = table.T
    tlin = _transpose_table(tt, tt).reshape(2 * _KPAIR, _D)
    ls = [
        f.T.reshape(_SR, 8, _B // 128, 128).transpose(0, 2, 1, 3)
        for f in (idx_f0, idx_f1, idx_f2, idx_f3)
    ]
    rows = _lookup(tlin, *ls)
    return (
        rows.reshape(_B, _SR, _F // 2, 8, 2 * _D)
        .transpose(0, 1, 3, 2, 4)
        .reshape(_B, _S, _F * _D)
    )


# speedup vs baseline: 2.0156x; 1.1621x over previous
import functools

import jax
import jax.numpy as jnp
from jax import lax
from jax.experimental import pallas as pl
from jax.experimental.pallas import tpu as pltpu
from jax.experimental.pallas import tpu_sc as plsc

_VOCAB = 1000000
_D = 64
_B = 1024
_S = 200
_F = 4

_NC = 2
_NS = 16
_NW = _NC * _NS
_TOTAL = _B * _S * _F
_PER_W = _TOTAL // _NW
_CHUNK = 128
_NBUF = 5
_STEPS = _PER_W // _CHUNK
_G = _STEPS // _NBUF

_BW = _B // _NW
_SR = _S // 8


_TBLK = 4096
_KPAIR = 503808
_NTB = _KPAIR // _TBLK


def _tp_body(ta_ref, tb_ref, out_ref):
    out_ref[:, 0:_D] = ta_ref[...].T
    out_ref[:, _D:128] = tb_ref[...].T


_transpose_table = pl.pallas_call(
    _tp_body,
    grid=(_NTB,),
    in_specs=[
        pl.BlockSpec((_D, _TBLK), lambda i: (0, i)),
        pl.BlockSpec(
            (_D, _TBLK),
            lambda i: (0, jnp.minimum(i + _NTB, (_VOCAB - 1) // _TBLK)),
        ),
    ],
    out_specs=pl.BlockSpec((_TBLK, 128), lambda i: (i, 0)),
    out_shape=jax.ShapeDtypeStruct((_KPAIR, 128), jnp.float32),
)


_mesh = plsc.VectorSubcoreMesh(core_axis_name="c", subcore_axis_name="s")


@functools.partial(
    pl.kernel,
    mesh=_mesh,
    out_type=jax.ShapeDtypeStruct((_TOTAL, _D), jnp.float32),
    scratch_types=[
        pltpu.VMEM((_SR, 8, _BW), jnp.int32),
        pltpu.VMEM((_SR, 8, _BW), jnp.int32),
        pltpu.VMEM((_SR, 8, _BW), jnp.int32),
        pltpu.VMEM((_SR, 8, _BW), jnp.int32),
        pltpu.VMEM((_PER_W,), jnp.int32),
        pltpu.VMEM((_NBUF, _CHUNK, _D), jnp.float32),
        pltpu.SemaphoreType.DMA((_NBUF,)),
        pltpu.SemaphoreType.DMA((_NBUF,)),
    ],
    compiler_params=pltpu.CompilerParams(use_tc_tiling_on_sc=False),
)
def _lookup(table_hbm, l0, l1, l2, l3, out_hbm,
            f0v, f1v, f2v, f3v, glist, rows_v, gsem, osem):
    wid = lax.axis_index("s") * _NC + lax.axis_index("c")
    base = wid * _PER_W
    cblk = wid // 4
    coff = (wid % 4) * _BW

    for l_hbm, f_v in ((l0, f0v), (l1, f1v), (l2, f2v), (l3, f3v)):
        pltpu.sync_copy(l_hbm.at[:, cblk, :, pl.ds(coff, _BW)], f_v)

    iot = lax.iota(jnp.int32, 16)
    lane_e = [iot == (2 * u) for u in range(8)]
    lane_o = [iot == (2 * u + 1) for u in range(8)]

    def rb_loop(t, carry):
        r = lax.shift_right_logical(t, 1)
        bh16 = (t & 1) * 16
        for cp in range(2):
            fe, fo = (f0v, f1v) if cp == 0 else (f2v, f3v)
            ve = [fe[r, u, pl.ds(bh16, 16)] for u in range(8)]
            vo = [fo[r, u, pl.ds(bh16, 16)] for u in range(8)]
            for bl in range(16):
                cidx = jnp.full((16,), bl, jnp.int32)
                acc = jnp.zeros((16,), jnp.int32)
                for u in range(8):
                    acc = jnp.where(lane_e[u], ve[u][cidx], acc)
                    acc = jnp.where(lane_o[u], vo[u][cidx], acc)
                g = acc + acc - jnp.where(acc >= _KPAIR, 2 * _KPAIR - 1, 0)
                off = ((bh16 + bl) * _SR + r) * 32 + cp * 16
                glist[pl.ds(off, 16)] = g
        return carry

    lax.fori_loop(0, 2 * _SR, rb_loop, 0)

    def fire_gather(j, slot):
        pltpu.async_copy(
            table_hbm.at[glist.at[pl.ds(j * _CHUNK, _CHUNK)]],
            rows_v.at[slot],
            gsem.at[slot],
        )

    def wait_gather(slot):
        pltpu.make_async_copy(
            table_hbm.at[glist.at[pl.ds(0, _CHUNK)]],
            rows_v.at[slot],
            gsem.at[slot],
        ).wait()

    def fire_store(j, slot):
        pltpu.async_copy(
            rows_v.at[slot],
            out_hbm.at[pl.ds(base + j * _CHUNK, _CHUNK)],
            osem.at[slot],
        )

    def wait_store(slot):
        pltpu.make_async_copy(
            rows_v.at[slot],
            out_hbm.at[pl.ds(base, _CHUNK)],
            osem.at[slot],
        ).wait()

    for b in range(_NBUF - 1):
        fire_gather(b, b)

    def run_step(j, b, first, fire):
        wait_gather(b)
        fire_store(j, b)
        if not first:
            wait_store((b - 1) % _NBUF)
        if fire:
            fire_gather(j + _NBUF - 1, (b - 1) % _NBUF)

    for b in range(_NBUF):
        run_step(b, b, first=(b == 0), fire=True)

    def outer(g, carry):
        for b in range(_NBUF):
            run_step(g * _NBUF + b, b, first=False, fire=True)
        return carry

    lax.fori_loop(1, _G - 1, outer, 0)

    for b in range(_NBUF):
        j = (_G - 1) * _NBUF + b
        run_step(j, b, first=False, fire=(j + _NBUF - 1 <= _STEPS - 1))
    wait_store(_NBUF - 1)


def kernel(table, idx_f0, idx_f1, idx_f2, idx_f3, seq_lens):
    del seq_lens
    tt = table.T
    tlin = _transpose_table(tt, tt).reshape(2 * _KPAIR, _D)
    ls = [
        f.T.reshape(_SR, 8, _B // 128, 128).transpose(0, 2, 1, 3)
        for f in (idx_f0, idx_f1, idx_f2, idx_f3)
    ]
    rows = _lookup(tlin, *ls)
    return (
        rows.reshape(_B, _SR, _F // 2, 8, 2 * _D)
        .transpose(0, 1, 3, 2, 4)
        .reshape(_B, _S, _F * _D)
    )


# speedup vs baseline: 2.1790x; 1.0811x over previous
import functools

import jax
import jax.numpy as jnp
from jax import lax
from jax.experimental import pallas as pl
from jax.experimental.pallas import tpu as pltpu
from jax.experimental.pallas import tpu_sc as plsc

_VOCAB = 1000000
_D = 64
_B = 1024
_S = 200
_F = 4

_NC = 2
_NS = 16
_NW = _NC * _NS
_TOTAL = _B * _S * _F
_PER_W = _TOTAL // _NW
_CHUNK = 128
_NBUF = 5
_STEPS = _PER_W // _CHUNK
_G = _STEPS // _NBUF

_BW = _B // _NW
_SR = _S // 8


_TBLK = 8192
_KPAIR = 507904
_NTB = _KPAIR // _TBLK


def _tp_body(ta_ref, tb_ref, out_ref):
    out_ref[:, 0:_D] = ta_ref[...].T
    out_ref[:, _D:128] = tb_ref[...].T


_transpose_table = pl.pallas_call(
    _tp_body,
    grid=(_NTB,),
    in_specs=[
        pl.BlockSpec((_D, _TBLK), lambda i: (0, i)),
        pl.BlockSpec(
            (_D, _TBLK),
            lambda i: (0, jnp.minimum(i + _NTB, (_VOCAB - 1) // _TBLK)),
        ),
    ],
    out_specs=pl.BlockSpec((_TBLK, 128), lambda i: (i, 0)),
    out_shape=jax.ShapeDtypeStruct((_KPAIR, 128), jnp.float32),
)


_mesh = plsc.VectorSubcoreMesh(core_axis_name="c", subcore_axis_name="s")


@functools.partial(
    pl.kernel,
    mesh=_mesh,
    out_type=jax.ShapeDtypeStruct((_TOTAL, _D), jnp.float32),
    scratch_types=[
        pltpu.VMEM((_SR, 8, _BW), jnp.int32),
        pltpu.VMEM((_SR, 8, _BW), jnp.int32),
        pltpu.VMEM((_SR, 8, _BW), jnp.int32),
        pltpu.VMEM((_SR, 8, _BW), jnp.int32),
        pltpu.VMEM((_PER_W,), jnp.int32),
        pltpu.VMEM((_NBUF, _CHUNK, _D), jnp.float32),
        pltpu.SemaphoreType.DMA((_NBUF,)),
        pltpu.SemaphoreType.DMA((_NBUF,)),
    ],
    compiler_params=pltpu.CompilerParams(use_tc_tiling_on_sc=False),
)
def _lookup(table_hbm, l0, l1, l2, l3, out_hbm,
            f0v, f1v, f2v, f3v, glist, rows_v, gsem, osem):
    wid = lax.axis_index("s") * _NC + lax.axis_index("c")
    base = wid * _PER_W
    cblk = wid // 4
    coff = (wid % 4) * _BW

    for l_hbm, f_v in ((l0, f0v), (l1, f1v), (l2, f2v), (l3, f3v)):
        pltpu.sync_copy(l_hbm.at[:, cblk, :, pl.ds(coff, _BW)], f_v)

    iot = lax.iota(jnp.int32, 16)
    lane_e = [iot == (2 * u) for u in range(8)]
    lane_o = [iot == (2 * u + 1) for u in range(8)]

    def rb_loop(t, carry):
        r = lax.shift_right_logical(t, 1)
        bh16 = (t & 1) * 16
        for cp in range(2):
            fe, fo = (f0v, f1v) if cp == 0 else (f2v, f3v)
            ve = [fe[r, u, pl.ds(bh16, 16)] for u in range(8)]
            vo = [fo[r, u, pl.ds(bh16, 16)] for u in range(8)]
            for bl in range(16):
                cidx = jnp.full((16,), bl, jnp.int32)
                acc = jnp.zeros((16,), jnp.int32)
                for u in range(8):
                    acc = jnp.where(lane_e[u], ve[u][cidx], acc)
                    acc = jnp.where(lane_o[u], vo[u][cidx], acc)
                g = acc + acc - jnp.where(acc >= _KPAIR, 2 * _KPAIR - 1, 0)
                off = ((bh16 + bl) * _SR + r) * 32 + cp * 16
                glist[pl.ds(off, 16)] = g
        return carry

    lax.fori_loop(0, 2 * _SR, rb_loop, 0)

    def fire_gather(j, slot):
        pltpu.async_copy(
            table_hbm.at[glist.at[pl.ds(j * _CHUNK, _CHUNK)]],
            rows_v.at[slot],
            gsem.at[slot],
        )

    def wait_gather(slot):
        pltpu.make_async_copy(
            table_hbm.at[glist.at[pl.ds(0, _CHUNK)]],
            rows_v.at[slot],
            gsem.at[slot],
        ).wait()

    def fire_store(j, slot):
        pltpu.async_copy(
            rows_v.at[slot],
            out_hbm.at[pl.ds(base + j * _CHUNK, _CHUNK)],
            osem.at[slot],
        )

    def wait_store(slot):
        pltpu.make_async_copy(
            rows_v.at[slot],
            out_hbm.at[pl.ds(base, _CHUNK)],
            osem.at[slot],
        ).wait()

    for b in range(_NBUF - 1):
        fire_gather(b, b)

    def run_step(j, b, first, fire):
        wait_gather(b)
        fire_store(j, b)
        if not first:
            wait_store((b - 1) % _NBUF)
        if fire:
            fire_gather(j + _NBUF - 1, (b - 1) % _NBUF)

    for b in range(_NBUF):
        run_step(b, b, first=(b == 0), fire=True)

    def outer(g, carry):
        for b in range(_NBUF):
            run_step(g * _NBUF + b, b, first=False, fire=True)
        return carry

    lax.fori_loop(1, _G - 1, outer, 0)

    for b in range(_NBUF):
        j = (_G - 1) * _NBUF + b
        run_step(j, b, first=False, fire=(j + _NBUF - 1 <= _STEPS - 1))
    wait_store(_NBUF - 1)


def kernel(table, idx_f0, idx_f1, idx_f2, idx_f3, seq_lens):
    del seq_lens
    tt = table.T
    tlin = _transpose_table(tt, tt).reshape(2 * _KPAIR, _D)
    ls = [
        f.T.reshape(_SR, 8, _B // 128, 128).transpose(0, 2, 1, 3)
        for f in (idx_f0, idx_f1, idx_f2, idx_f3)
    ]
    rows = _lookup(tlin, *ls)
    return (
        rows.reshape(_B, _SR, _F // 2, 8, 2 * _D)
        .transpose(0, 1, 3, 2, 4)
        .reshape(_B, _S, _F * _D)
    )
